# Initial kernel scaffold; baseline (speedup 1.0000x reference)
import jax
import jax.numpy as jnp
from jax.experimental import pallas as pl


def kernel(x, Ws, bs, Wf, bf, Wo, bo):
    raise NotImplementedError("write your pallas kernel here")


# speedup vs baseline: 3.9604x; 3.9604x over previous
import functools

import jax
import jax.numpy as jnp
from jax import lax
from jax.experimental import pallas as pl
from jax.experimental.pallas import tpu as pltpu

_K = 40


def _gravnet_body(x_rows_ref, x_full_ref, Ws_ref, bs_ref, Wf_ref, bf_ref,
                  Wo_ref, bo_ref, out_ref, *, R, V, F, K):
    xr = x_rows_ref[0]
    xa = x_full_ref[0]
    Ws = Ws_ref[...]
    Wf = Wf_ref[...]
    bs = bs_ref[...]
    bf = bf_ref[...]

    ca = jnp.dot(xa, Ws, preferred_element_type=jnp.float32) + bs
    fa = jnp.dot(xa, Wf, preferred_element_type=jnp.float32) + bf
    cr = jnp.dot(xr, Ws, preferred_element_type=jnp.float32) + bs

    a2 = jnp.sum(cr * cr, axis=1, keepdims=True)
    b2 = jnp.sum(ca * ca, axis=1, keepdims=True)
    cross = lax.dot_general(cr, ca, (((1,), (1,)), ((), ())),
                            preferred_element_type=jnp.float32)
    dm = a2 - 2.0 * cross + b2.reshape(1, V)

    iota = lax.broadcasted_iota(jnp.int32, (R, V), 1)
    big = jnp.float32(jnp.inf)
    P = fa.shape[1]

    def extract(dm):
        m = jnp.min(dm, axis=1, keepdims=True)
        cand = jnp.where(dm == m, iota, V)
        idx = jnp.min(cand, axis=1, keepdims=True)
        onehot = iota == idx
        dm = jnp.where(onehot, big, dm)
        return dm, m, onehot

    dm, _, _ = extract(dm)

    def step(_, carry):
        dm, mx, sm = carry
        dm, m, onehot = extract(dm)
        w = jnp.exp(-jnp.abs(m * 10.0))
        nf = jnp.dot(onehot.astype(jnp.float32), fa,
                     preferred_element_type=jnp.float32)
        wf = nf * w
        return dm, jnp.maximum(mx, wf), sm + wf

    mx0 = jnp.full((R, P), -big, dtype=jnp.float32)
    sm0 = jnp.zeros((R, P), dtype=jnp.float32)
    _, mx, sm = lax.fori_loop(0, K - 1, step, (dm, mx0, sm0))
    mean = sm / jnp.float32(K - 1)

    Wo = Wo_ref[...]
    bo = bo_ref[...]
    acc = jnp.dot(xr, Wo[:F], preferred_element_type=jnp.float32)
    acc += jnp.dot(mx, Wo[F:F + P], preferred_element_type=jnp.float32)
    acc += jnp.dot(mean, Wo[F + P:], preferred_element_type=jnp.float32)
    out_ref[0] = jnp.tanh(acc + bo)


def kernel(x, Ws, bs, Wf, bf, Wo, bo):
    B, V, F = x.shape
    D = Ws.shape[1]
    P = Wf.shape[1]
    O = Wo.shape[1]
    R = min(256, V)

    bs2 = bs.reshape(1, D)
    bf2 = bf.reshape(1, P)
    bo2 = bo.reshape(1, O)

    grid = (B, V // R)
    body = functools.partial(_gravnet_body, R=R, V=V, F=F, K=_K)
    return pl.pallas_call(
        body,
        grid=grid,
        in_specs=[
            pl.BlockSpec((1, R, F), lambda b, i: (b, i, 0)),
            pl.BlockSpec((1, V, F), lambda b, i: (b, 0, 0)),
            pl.BlockSpec((F, D), lambda b, i: (0, 0)),
            pl.BlockSpec((1, D), lambda b, i: (0, 0)),
            pl.BlockSpec((F, P), lambda b, i: (0, 0)),
            pl.BlockSpec((1, P), lambda b, i: (0, 0)),
            pl.BlockSpec((F + 2 * P, O), lambda b, i: (0, 0)),
            pl.BlockSpec((1, O), lambda b, i: (0, 0)),
        ],
        out_specs=pl.BlockSpec((1, R, O), lambda b, i: (b, i, 0)),
        out_shape=jax.ShapeDtypeStruct((B, V, O), jnp.float32),
    )(x, x, Ws, bs2, Wf, bf2, Wo, bo2)
